# baseline (speedup 1.0000x reference)
import functools

import jax
import jax.numpy as jnp
from jax import lax
from jax.experimental import pallas as pl
from jax.experimental.pallas import tpu as pltpu
from jax.experimental.pallas import tpu_sc as plsc

N = 10000
E = 320000
D_IN = 128
D_HID = 128
D_CLS = 40
D_CLS_PAD = 128

NC = 2
NS = 16
NW = NC * NS

CHUNK = 128
NB = 2
NWIN = 2
NCHUNK = (NB * NWIN) * (-(-E // (NW * CHUNK * NB * NWIN)))
CPW = NCHUNK // NWIN
EPW = NCHUNK * CHUNK
E_PAD = EPW * NW

N_ACC = 10240
ROWS_PER_TILE = N_ACC // NS
R_BLK = 128

NCH0 = 96
NCH1 = 2 * NCHUNK - NCH0
CPW0 = NCH0 // NWIN
CPW1 = NCH1 // NWIN
CPW_MAX = max(CPW0, CPW1)


def _sc_mesh():
    return plsc.VectorSubcoreMesh(core_axis_name="c", subcore_axis_name="s")


def _deg_kernel(src_h, dst_h, out0, out1, dst_i, ones_v, zero_v, dacc,
                ss0, ss1):
    ssem = (ss0, ss1)
    c = lax.axis_index("c")
    s = lax.axis_index("s")
    wid = c * NS + s

    def fill(i, _):
        r = i // 8
        k = i % 8
        ones_v[r, pl.ds(k * 16, 16)] = jnp.ones((16,), jnp.float32)
        zero_v[r, pl.ds(k * 16, 16)] = jnp.zeros((16,), jnp.float32)
        return 0
    lax.fori_loop(0, CHUNK * 8, fill, 0)

    def zacc(i, _):
        pltpu.sync_copy(zero_v, dacc.at[pl.ds(s * ROWS_PER_TILE + i * CHUNK, CHUNK)])
        return 0
    lax.fori_loop(0, ROWS_PER_TILE // CHUNK, zacc, 0)
    pltpu.sync_copy(dst_h.at[pl.ds(wid * NCHUNK, NCHUNK)], dst_i)
    plsc.subcore_barrier()

    def grp(g, _):
        for b in range(NB):
            j = g * NB + b

            @pl.when(g > 0)
            def _():
                pltpu.make_async_copy(ones_v, dacc.at[dst_i.at[j]], ssem[b]).wait()
            pltpu.async_copy(ones_v, dacc.at[dst_i.at[j]], ssem[b], add=True)
        return 0
    lax.fori_loop(0, NCHUNK // NB, grp, 0)
    for b in range(NB):
        pltpu.make_async_copy(ones_v, dacc.at[dst_i.at[0]], ssem[b]).wait()
    plsc.subcore_barrier()

    def wout(i, _):
        r0 = s * ROWS_PER_TILE + i * CHUNK

        @pl.when(c == 0)
        def _():
            pltpu.sync_copy(dacc.at[pl.ds(r0, CHUNK)], out0.at[pl.ds(r0, CHUNK)])

        @pl.when(c == 1)
        def _():
            pltpu.sync_copy(dacc.at[pl.ds(r0, CHUNK)], out1.at[pl.ds(r0, CHUNK)])
        return 0
    lax.fori_loop(0, ROWS_PER_TILE // CHUNK, wout, 0)


def _sc_degree(src_e, dst_e):
    kfn = pl.kernel(
        _deg_kernel,
        out_type=(
            jax.ShapeDtypeStruct((N_ACC, 128), jnp.float32),
            jax.ShapeDtypeStruct((N_ACC, 128), jnp.float32),
        ),
        mesh=_sc_mesh(),
        scratch_types=[
            pltpu.VMEM((NCHUNK, CHUNK), jnp.int32),
            pltpu.VMEM((CHUNK, 128), jnp.float32),
            pltpu.VMEM((CHUNK, 128), jnp.float32),
            pltpu.VMEM_SHARED((N_ACC, 128), jnp.float32),
            pltpu.SemaphoreType.DMA,
            pltpu.SemaphoreType.DMA,
        ],
    )
    return kfn(src_e, dst_e)


def _prop_body(d, table_h, src_h, dst_h, out0, out1,
               src_i, dst_i, rows0, rows1, acc, gs0, gs1, ss0, ss1):
    rows = (rows0, rows1)
    gsem = (gs0, gs1)
    ssem = (ss0, ss1)
    c = lax.axis_index("c")
    s = lax.axis_index("s")
    wid = c * NS + s
    vregs_per_row = d // 16

    def zrow(i, _):
        r = i // vregs_per_row
        k = i % vregs_per_row
        rows0[r, pl.ds(k * 16, 16)] = jnp.zeros((16,), jnp.float32)
        return 0
    lax.fori_loop(0, CHUNK * vregs_per_row, zrow, 0)

    def zacc(i, _):
        pltpu.sync_copy(rows0, acc.at[pl.ds(s * ROWS_PER_TILE + i * CHUNK, CHUNK)])
        return 0
    lax.fori_loop(0, ROWS_PER_TILE // CHUNK, zacc, 0)
    plsc.subcore_barrier()

    row_base = jnp.where(c == 0, s * NCH0, NS * NCH0 + s * NCH1)
    cpw = jnp.where(c == 0, CPW0, CPW1)

    def win(w, _):
        w0 = row_base + w * cpw
        pltpu.sync_copy(src_h.at[pl.ds(w0, CPW_MAX)], src_i)
        pltpu.sync_copy(dst_h.at[pl.ds(w0, CPW_MAX)], dst_i)
        for b in range(NB):
            pltpu.async_copy(table_h.at[src_i.at[b]], rows[b], gsem[b])

        def grp(g, _):
            for b in range(NB):
                j = g * NB + b
                pltpu.make_async_copy(table_h.at[src_i.at[j]], rows[b],
                                      gsem[b]).wait()
                pltpu.async_copy(rows[b], acc.at[dst_i.at[j]], ssem[b], add=True)
                jn = j + NB

                @pl.when(jn < cpw)
                def _():
                    pltpu.make_async_copy(rows[b], acc.at[dst_i.at[j]],
                                          ssem[b]).wait()
                    pltpu.async_copy(table_h.at[src_i.at[jn]], rows[b], gsem[b])
            return 0
        lax.fori_loop(0, cpw // NB, grp, 0)
        for b in range(NB):
            pltpu.make_async_copy(rows[b], acc.at[dst_i.at[0]], ssem[b]).wait()
        return 0
    lax.fori_loop(0, NWIN, win, 0)
    plsc.subcore_barrier()

    def wout(i, _):
        r0 = s * ROWS_PER_TILE + i * CHUNK

        @pl.when(c == 0)
        def _():
            pltpu.sync_copy(acc.at[pl.ds(r0, CHUNK)], out0.at[pl.ds(r0, CHUNK)])

        @pl.when(c == 1)
        def _():
            pltpu.sync_copy(acc.at[pl.ds(r0, CHUNK)], out1.at[pl.ds(r0, CHUNK)])
        return 0
    lax.fori_loop(0, ROWS_PER_TILE // CHUNK, wout, 0)


def _sc_prop(table, src_e, dst_e, d):
    kfn = pl.kernel(
        functools.partial(_prop_body, d),
        out_type=(
            jax.ShapeDtypeStruct((N_ACC, d), jnp.float32),
            jax.ShapeDtypeStruct((N_ACC, d), jnp.float32),
        ),
        mesh=_sc_mesh(),
        scratch_types=(
            [
                pltpu.VMEM((CPW_MAX, CHUNK), jnp.int32),
                pltpu.VMEM((CPW_MAX, CHUNK), jnp.int32),
            ]
            + [pltpu.VMEM((CHUNK, d), jnp.float32) for _ in range(NB)]
            + [pltpu.VMEM_SHARED((N_ACC, d), jnp.float32)]
            + [pltpu.SemaphoreType.DMA for _ in range(2 * NB)]
        ),
    )
    return kfn(table, src_e, dst_e)


def _row_grid(n):
    return -(-n // R_BLK)


def _mm_scale_body(x_ref, w_ref, p0_ref, p1_ref, o_ref, norm_ref):
    deg = jnp.sum(p0_ref[...] + p1_ref[...], axis=1, keepdims=True) * (1.0 / 128.0)
    nrm = jnp.where(deg > 0.0, lax.rsqrt(jnp.maximum(deg, 1.0)), 0.0)
    norm_ref[...] = nrm
    o_ref[...] = jnp.dot(x_ref[...], w_ref[...],
                         preferred_element_type=jnp.float32) * nrm


def _tc_in_proj(x, w0, degp0, degp1):
    return pl.pallas_call(
        _mm_scale_body,
        grid=(_row_grid(N),),
        in_specs=[
            pl.BlockSpec((R_BLK, D_IN), lambda i: (i, 0)),
            pl.BlockSpec((D_IN, D_HID), lambda i: (0, 0)),
            pl.BlockSpec((R_BLK, 128), lambda i: (i, 0)),
            pl.BlockSpec((R_BLK, 128), lambda i: (i, 0)),
        ],
        out_specs=(
            pl.BlockSpec((R_BLK, D_HID), lambda i: (i, 0)),
            pl.BlockSpec((R_BLK, 1), lambda i: (i, 0)),
        ),
        out_shape=(
            jax.ShapeDtypeStruct((N, D_HID), jnp.float32),
            jax.ShapeDtypeStruct((_row_grid(N) * R_BLK, 1), jnp.float32),
        ),
    )(x, w0, degp0, degp1)


def _relu_scale_body(p0_ref, p1_ref, norm_ref, b_ref, o_ref):
    nrm = norm_ref[...]
    t = (p0_ref[...] + p1_ref[...]) * nrm + b_ref[...][None, :]
    o_ref[...] = jnp.maximum(t, 0.0) * nrm


def _tc_relu_scale(p0, p1, norm, b):
    return pl.pallas_call(
        _relu_scale_body,
        grid=(_row_grid(N),),
        in_specs=[
            pl.BlockSpec((R_BLK, D_HID), lambda i: (i, 0)),
            pl.BlockSpec((R_BLK, D_HID), lambda i: (i, 0)),
            pl.BlockSpec((R_BLK, 1), lambda i: (i, 0)),
            pl.BlockSpec((D_HID,), lambda i: (0,)),
        ],
        out_specs=pl.BlockSpec((R_BLK, D_HID), lambda i: (i, 0)),
        out_shape=jax.ShapeDtypeStruct((N, D_HID), jnp.float32),
    )(p0, p1, norm, b)


def _invdeg_body(p0_ref, p1_ref, norm_ref, o_ref):
    nrm = norm_ref[...]
    o_ref[...] = (p0_ref[...] + p1_ref[...]) * (nrm * nrm)


def _tc_invdeg_scale(p0, p1, norm):
    return pl.pallas_call(
        _invdeg_body,
        grid=(_row_grid(N),),
        in_specs=[
            pl.BlockSpec((R_BLK, D_HID), lambda i: (i, 0)),
            pl.BlockSpec((R_BLK, D_HID), lambda i: (i, 0)),
            pl.BlockSpec((R_BLK, 1), lambda i: (i, 0)),
        ],
        out_specs=pl.BlockSpec((R_BLK, D_HID), lambda i: (i, 0)),
        out_shape=jax.ShapeDtypeStruct((N, D_HID), jnp.float32),
    )(p0, p1, norm)


def _mid_body(p0_ref, p1_ref, norm_ref, w1_ref, b1_ref, w2_ref, o_ref):
    nrm = norm_ref[...]
    t = (p0_ref[...] + p1_ref[...]) * nrm
    h = jnp.dot(t, w1_ref[...], preferred_element_type=jnp.float32)
    h = jnp.maximum(h + b1_ref[...][None, :], 0.0)
    z = jnp.dot(h, w2_ref[...], preferred_element_type=jnp.float32)
    o_ref[...] = z * nrm


def _tc_mid(p0, p1, norm, w1, b1, w2pad):
    return pl.pallas_call(
        _mid_body,
        grid=(_row_grid(N),),
        in_specs=[
            pl.BlockSpec((R_BLK, D_HID), lambda i: (i, 0)),
            pl.BlockSpec((R_BLK, D_HID), lambda i: (i, 0)),
            pl.BlockSpec((R_BLK, 1), lambda i: (i, 0)),
            pl.BlockSpec((D_HID, D_HID), lambda i: (0, 0)),
            pl.BlockSpec((D_HID,), lambda i: (0,)),
            pl.BlockSpec((D_HID, D_CLS_PAD), lambda i: (0, 0)),
        ],
        out_specs=pl.BlockSpec((R_BLK, D_CLS_PAD), lambda i: (i, 0)),
        out_shape=jax.ShapeDtypeStruct((N, D_CLS_PAD), jnp.float32),
    )(p0, p1, norm, w1, b1, w2pad)


def _final_body(q0_ref, q1_ref, norm_ref, b2_ref, o_ref):
    o_ref[...] = ((q0_ref[...] + q1_ref[...]) * norm_ref[...]
                  + b2_ref[...][None, :])


def _tc_final(q0, q1, norm, b2pad):
    return pl.pallas_call(
        _final_body,
        grid=(_row_grid(N),),
        in_specs=[
            pl.BlockSpec((R_BLK, D_CLS_PAD), lambda i: (i, 0)),
            pl.BlockSpec((R_BLK, D_CLS_PAD), lambda i: (i, 0)),
            pl.BlockSpec((R_BLK, 1), lambda i: (i, 0)),
            pl.BlockSpec((D_CLS_PAD,), lambda i: (0,)),
        ],
        out_specs=pl.BlockSpec((R_BLK, D_CLS_PAD), lambda i: (i, 0)),
        out_shape=jax.ShapeDtypeStruct((N, D_CLS_PAD), jnp.float32),
    )(q0, q1, norm, b2pad)


def kernel(x, edge_index, W0, b0, W1, b1, W2, b2):
    src = edge_index[0]
    dst = edge_index[1]
    n_dummy = E_PAD - E
    src_e = jnp.concatenate([src, jnp.zeros((n_dummy,), jnp.int32)])
    dst_e = jnp.concatenate(
        [dst, N + (jnp.arange(n_dummy, dtype=jnp.int32) % (N_ACC - N))])
    src_e = src_e.reshape(NW * NCHUNK, CHUNK)
    dst_e = dst_e.reshape(NW * NCHUNK, CHUNK)

    degp0, degp1 = _sc_degree(src_e, dst_e)
    table1, norm = _tc_in_proj(x, W0, degp0, degp1)
    p0, p1 = _sc_prop(table1, src_e, dst_e, D_HID)

    g1 = _tc_relu_scale(p0, p1, norm, b0)
    p0, p1 = _sc_prop(g1, src_e, dst_e, D_HID)

    u = _tc_invdeg_scale(p0, p1, norm)
    p0, p1 = _sc_prop(u, src_e, dst_e, D_HID)

    w2pad = jnp.pad(W2, ((0, 0), (0, D_CLS_PAD - D_CLS)))
    table4 = _tc_mid(p0, p1, norm, W1, b1, w2pad)
    q0, q1 = _sc_prop(table4, src_e, dst_e, D_CLS_PAD)

    b2pad = jnp.pad(b2, (0, D_CLS_PAD - D_CLS))
    out = _tc_final(q0, q1, norm, b2pad)
    return out[:, :D_CLS]


# speedup vs baseline: 1.0924x; 1.0924x over previous
import functools

import jax
import jax.numpy as jnp
from jax import lax
from jax.experimental import pallas as pl
from jax.experimental.pallas import tpu as pltpu
from jax.experimental.pallas import tpu_sc as plsc

N = 10000
E = 320000
D_IN = 128
D_HID = 128
D_CLS = 40
D_CLS_PAD = 128

NC = 2
NS = 16
NW = NC * NS

CHUNK = 128
NB = 2
NWIN = 2
NCHUNK = (NB * NWIN) * (-(-E // (NW * CHUNK * NB * NWIN)))
CPW = NCHUNK // NWIN
EPW = NCHUNK * CHUNK
E_PAD = EPW * NW

N_ACC = 10240
ROWS_PER_TILE = N_ACC // NS
R_BLK = 128

NCH0 = 96
NCH1 = 2 * NCHUNK - NCH0
CPW0 = NCH0 // NWIN
CPW1 = NCH1 // NWIN
CPW_MAX = max(CPW0, CPW1)


def _sc_mesh():
    return plsc.VectorSubcoreMesh(core_axis_name="c", subcore_axis_name="s")


def _deg_kernel(src_h, dst_h, out0, out1, dst_i, ones_v, zero_v, dacc,
                ss0, ss1):
    ssem = (ss0, ss1)
    c = lax.axis_index("c")
    s = lax.axis_index("s")
    wid = c * NS + s

    def fill(i, _):
        r = i // 8
        k = i % 8
        ones_v[r, pl.ds(k * 16, 16)] = jnp.ones((16,), jnp.float32)
        zero_v[r, pl.ds(k * 16, 16)] = jnp.zeros((16,), jnp.float32)
        return 0
    lax.fori_loop(0, CHUNK * 8, fill, 0)

    def zacc(i, _):
        pltpu.sync_copy(zero_v, dacc.at[pl.ds(s * ROWS_PER_TILE + i * CHUNK, CHUNK)])
        return 0
    lax.fori_loop(0, ROWS_PER_TILE // CHUNK, zacc, 0)
    pltpu.sync_copy(dst_h.at[pl.ds(wid * NCHUNK, NCHUNK)], dst_i)
    plsc.subcore_barrier()

    def grp(g, _):
        for b in range(NB):
            j = g * NB + b

            @pl.when(g > 0)
            def _():
                pltpu.make_async_copy(ones_v, dacc.at[dst_i.at[j]], ssem[b]).wait()
            pltpu.async_copy(ones_v, dacc.at[dst_i.at[j]], ssem[b], add=True)
        return 0
    lax.fori_loop(0, NCHUNK // NB, grp, 0)
    for b in range(NB):
        pltpu.make_async_copy(ones_v, dacc.at[dst_i.at[0]], ssem[b]).wait()
    plsc.subcore_barrier()

    def wout(i, _):
        r0 = s * ROWS_PER_TILE + i * CHUNK

        @pl.when(c == 0)
        def _():
            pltpu.sync_copy(dacc.at[pl.ds(r0, CHUNK)], out0.at[pl.ds(r0, CHUNK)])

        @pl.when(c == 1)
        def _():
            pltpu.sync_copy(dacc.at[pl.ds(r0, CHUNK)], out1.at[pl.ds(r0, CHUNK)])
        return 0
    lax.fori_loop(0, ROWS_PER_TILE // CHUNK, wout, 0)


def _sc_degree(src_e, dst_e):
    kfn = pl.kernel(
        _deg_kernel,
        out_type=(
            jax.ShapeDtypeStruct((N_ACC, 128), jnp.float32),
            jax.ShapeDtypeStruct((N_ACC, 128), jnp.float32),
        ),
        mesh=_sc_mesh(),
        scratch_types=[
            pltpu.VMEM((NCHUNK, CHUNK), jnp.int32),
            pltpu.VMEM((CHUNK, 128), jnp.float32),
            pltpu.VMEM((CHUNK, 128), jnp.float32),
            pltpu.VMEM_SHARED((N_ACC, 128), jnp.float32),
            pltpu.SemaphoreType.DMA,
            pltpu.SemaphoreType.DMA,
        ],
    )
    return kfn(src_e, dst_e)


def _prop_body(d, table_h, src_h, dst_h, out0, out1,
               src_i, dst_i, rows0, rows1, acc, gs0, gs1, ss0, ss1):
    rows = (rows0, rows1)
    gsem = (gs0, gs1)
    ssem = (ss0, ss1)
    c = lax.axis_index("c")
    s = lax.axis_index("s")
    wid = c * NS + s
    vregs_per_row = d // 16

    def zrow(i, _):
        r = i // vregs_per_row
        k = i % vregs_per_row
        rows0[r, pl.ds(k * 16, 16)] = jnp.zeros((16,), jnp.float32)
        return 0
    lax.fori_loop(0, CHUNK * vregs_per_row, zrow, 0)

    def zacc(i, _):
        pltpu.sync_copy(rows0, acc.at[pl.ds(s * ROWS_PER_TILE + i * CHUNK, CHUNK)])
        return 0
    lax.fori_loop(0, ROWS_PER_TILE // CHUNK, zacc, 0)
    plsc.subcore_barrier()

    row_base = jnp.where(c == 0, s * NCH0, NS * NCH0 + s * NCH1)
    cpw = jnp.where(c == 0, CPW0, CPW1)

    def win(w, _):
        w0 = row_base + w * cpw
        pltpu.sync_copy(src_h.at[pl.ds(w0, CPW_MAX)], src_i)
        pltpu.sync_copy(dst_h.at[pl.ds(w0, CPW_MAX)], dst_i)
        for b in range(NB):
            pltpu.async_copy(table_h.at[src_i.at[b]], rows[b], gsem[b])

        def grp(g, _):
            for b in range(NB):
                j = g * NB + b
                pltpu.make_async_copy(table_h.at[src_i.at[j]], rows[b],
                                      gsem[b]).wait()
                pltpu.async_copy(rows[b], acc.at[dst_i.at[j]], ssem[b], add=True)
                jn = j + NB

                @pl.when(jn < cpw)
                def _():
                    pltpu.make_async_copy(rows[b], acc.at[dst_i.at[j]],
                                          ssem[b]).wait()
                    pltpu.async_copy(table_h.at[src_i.at[jn]], rows[b], gsem[b])
            return 0
        lax.fori_loop(0, cpw // NB, grp, 0)
        for b in range(NB):
            pltpu.make_async_copy(rows[b], acc.at[dst_i.at[0]], ssem[b]).wait()
        return 0
    lax.fori_loop(0, NWIN, win, 0)
    plsc.subcore_barrier()

    def wout(i, _):
        r0 = s * ROWS_PER_TILE + i * CHUNK

        @pl.when(c == 0)
        def _():
            pltpu.sync_copy(acc.at[pl.ds(r0, CHUNK)], out0.at[pl.ds(r0, CHUNK)])

        @pl.when(c == 1)
        def _():
            pltpu.sync_copy(acc.at[pl.ds(r0, CHUNK)], out1.at[pl.ds(r0, CHUNK)])
        return 0
    lax.fori_loop(0, ROWS_PER_TILE // CHUNK, wout, 0)


def _sc_prop(table, src_e, dst_e, d):
    kfn = pl.kernel(
        functools.partial(_prop_body, d),
        out_type=(
            jax.ShapeDtypeStruct((N_ACC, d), jnp.float32),
            jax.ShapeDtypeStruct((N_ACC, d), jnp.float32),
        ),
        mesh=_sc_mesh(),
        scratch_types=(
            [
                pltpu.VMEM((CPW_MAX, CHUNK), jnp.int32),
                pltpu.VMEM((CPW_MAX, CHUNK), jnp.int32),
            ]
            + [pltpu.VMEM((CHUNK, d), jnp.float32) for _ in range(NB)]
            + [pltpu.VMEM_SHARED((N_ACC, d), jnp.float32)]
            + [pltpu.SemaphoreType.DMA for _ in range(2 * NB)]
        ),
    )
    return kfn(table, src_e, dst_e)


def _row_grid(n):
    return -(-n // R_BLK)


def _norm_body(p0_ref, p1_ref, norm_ref):
    deg = jnp.sum(p0_ref[...] + p1_ref[...], axis=1, keepdims=True) * (1.0 / 128.0)
    norm_ref[...] = jnp.where(deg > 0.0, lax.rsqrt(jnp.maximum(deg, 1.0)), 0.0)


def _tc_norm(degp0, degp1):
    return pl.pallas_call(
        _norm_body,
        grid=(N_ACC // R_BLK,),
        in_specs=[
            pl.BlockSpec((R_BLK, 128), lambda i: (i, 0)),
            pl.BlockSpec((R_BLK, 128), lambda i: (i, 0)),
        ],
        out_specs=pl.BlockSpec((R_BLK, 1), lambda i: (i, 0)),
        out_shape=jax.ShapeDtypeStruct((N_ACC, 1), jnp.float32),
    )(degp0, degp1)


def _mm_scale_body(x_ref, w_ref, norm_ref, o_ref):
    o_ref[...] = jnp.dot(x_ref[...], w_ref[...],
                         preferred_element_type=jnp.float32) * norm_ref[...]


def _tc_in_proj(x, w0, norm):
    return pl.pallas_call(
        _mm_scale_body,
        grid=(_row_grid(N),),
        in_specs=[
            pl.BlockSpec((R_BLK, D_IN), lambda i: (i, 0)),
            pl.BlockSpec((D_IN, D_HID), lambda i: (0, 0)),
            pl.BlockSpec((R_BLK, 1), lambda i: (i, 0)),
        ],
        out_specs=pl.BlockSpec((R_BLK, D_HID), lambda i: (i, 0)),
        out_shape=jax.ShapeDtypeStruct((N, D_HID), jnp.float32),
    )(x, w0, norm)


def _relu_scale_body(p0_ref, p1_ref, norm_ref, b_ref, o_ref):
    nrm = norm_ref[...]
    t = (p0_ref[...] + p1_ref[...]) * nrm + b_ref[...][None, :]
    o_ref[...] = jnp.maximum(t, 0.0) * nrm


def _tc_relu_scale(p0, p1, norm, b):
    return pl.pallas_call(
        _relu_scale_body,
        grid=(_row_grid(N),),
        in_specs=[
            pl.BlockSpec((R_BLK, D_HID), lambda i: (i, 0)),
            pl.BlockSpec((R_BLK, D_HID), lambda i: (i, 0)),
            pl.BlockSpec((R_BLK, 1), lambda i: (i, 0)),
            pl.BlockSpec((D_HID,), lambda i: (0,)),
        ],
        out_specs=pl.BlockSpec((R_BLK, D_HID), lambda i: (i, 0)),
        out_shape=jax.ShapeDtypeStruct((N, D_HID), jnp.float32),
    )(p0, p1, norm, b)


def _invdeg_body(p0_ref, p1_ref, norm_ref, o_ref):
    nrm = norm_ref[...]
    o_ref[...] = (p0_ref[...] + p1_ref[...]) * (nrm * nrm)


def _tc_invdeg_scale(p0, p1, norm):
    return pl.pallas_call(
        _invdeg_body,
        grid=(_row_grid(N),),
        in_specs=[
            pl.BlockSpec((R_BLK, D_HID), lambda i: (i, 0)),
            pl.BlockSpec((R_BLK, D_HID), lambda i: (i, 0)),
            pl.BlockSpec((R_BLK, 1), lambda i: (i, 0)),
        ],
        out_specs=pl.BlockSpec((R_BLK, D_HID), lambda i: (i, 0)),
        out_shape=jax.ShapeDtypeStruct((N, D_HID), jnp.float32),
    )(p0, p1, norm)


def _mid_body(p0_ref, p1_ref, norm_ref, w1_ref, b1_ref, w2_ref, o_ref):
    nrm = norm_ref[...]
    t = (p0_ref[...] + p1_ref[...]) * nrm
    h = jnp.dot(t, w1_ref[...], preferred_element_type=jnp.float32)
    h = jnp.maximum(h + b1_ref[...][None, :], 0.0)
    z = jnp.dot(h, w2_ref[...], preferred_element_type=jnp.float32)
    o_ref[...] = z * nrm


def _tc_mid(p0, p1, norm, w1, b1, w2pad):
    return pl.pallas_call(
        _mid_body,
        grid=(_row_grid(N),),
        in_specs=[
            pl.BlockSpec((R_BLK, D_HID), lambda i: (i, 0)),
            pl.BlockSpec((R_BLK, D_HID), lambda i: (i, 0)),
            pl.BlockSpec((R_BLK, 1), lambda i: (i, 0)),
            pl.BlockSpec((D_HID, D_HID), lambda i: (0, 0)),
            pl.BlockSpec((D_HID,), lambda i: (0,)),
            pl.BlockSpec((D_HID, D_CLS_PAD), lambda i: (0, 0)),
        ],
        out_specs=pl.BlockSpec((R_BLK, D_CLS_PAD), lambda i: (i, 0)),
        out_shape=jax.ShapeDtypeStruct((N, D_CLS_PAD), jnp.float32),
    )(p0, p1, norm, w1, b1, w2pad)


def _final_body(q0_ref, q1_ref, norm_ref, b2_ref, o_ref):
    o_ref[...] = ((q0_ref[...] + q1_ref[...]) * norm_ref[...]
                  + b2_ref[...][None, :])


def _tc_final(q0, q1, norm, b2pad):
    return pl.pallas_call(
        _final_body,
        grid=(_row_grid(N),),
        in_specs=[
            pl.BlockSpec((R_BLK, D_CLS_PAD), lambda i: (i, 0)),
            pl.BlockSpec((R_BLK, D_CLS_PAD), lambda i: (i, 0)),
            pl.BlockSpec((R_BLK, 1), lambda i: (i, 0)),
            pl.BlockSpec((D_CLS_PAD,), lambda i: (0,)),
        ],
        out_specs=pl.BlockSpec((R_BLK, D_CLS_PAD), lambda i: (i, 0)),
        out_shape=jax.ShapeDtypeStruct((N, D_CLS_PAD), jnp.float32),
    )(q0, q1, norm, b2pad)


def kernel(x, edge_index, W0, b0, W1, b1, W2, b2):
    src = edge_index[0]
    dst = edge_index[1]
    n_dummy = E_PAD - E
    src_e = jnp.concatenate([src, jnp.zeros((n_dummy,), jnp.int32)])
    dst_e = jnp.concatenate(
        [dst, N + (jnp.arange(n_dummy, dtype=jnp.int32) % (N_ACC - N))])
    src_e = src_e.reshape(NW * NCHUNK, CHUNK)
    dst_e = dst_e.reshape(NW * NCHUNK, CHUNK)

    degp0, degp1 = _sc_degree(src_e, dst_e)
    norm = _tc_norm(degp0, degp1)
    table1 = _tc_in_proj(x, W0, norm)
    p0, p1 = _sc_prop(table1, src_e, dst_e, D_HID)

    g1 = _tc_relu_scale(p0, p1, norm, b0)
    p0, p1 = _sc_prop(g1, src_e, dst_e, D_HID)

    u = _tc_invdeg_scale(p0, p1, norm)
    p0, p1 = _sc_prop(u, src_e, dst_e, D_HID)

    w2pad = jnp.pad(W2, ((0, 0), (0, D_CLS_PAD - D_CLS)))
    table4 = _tc_mid(p0, p1, norm, W1, b1, w2pad)
    q0, q1 = _sc_prop(table4, src_e, dst_e, D_CLS_PAD)

    b2pad = jnp.pad(b2, (0, D_CLS_PAD - D_CLS))
    out = _tc_final(q0, q1, norm, b2pad)
    return out[:, :D_CLS]
